# baseline (speedup 1.0000x reference)
import functools

import jax
import jax.numpy as jnp
from jax import lax
from jax.experimental import pallas as pl
from jax.experimental.pallas import tpu as pltpu
from jax.experimental.pallas import tpu_sc as plsc

N = 10000
D = 128
E = 320000
NP = 10240
CHUNK = 128
CPT = 80
EPT = CPT * CHUNK
EP = 32 * EPT
RPT = NP // 16


def _wid():
    cid = lax.axis_index("c")
    sid = lax.axis_index("s")
    return cid, sid, sid * 2 + cid


def _deg_body(dst3, degp, idxbuf, deg_local):
    cid, sid, wid = _wid()
    zeros16 = jnp.zeros((16,), jnp.float32)
    ones16 = jnp.ones((16,), jnp.float32)

    @pl.loop(0, NP // 16)
    def _(i):
        deg_local[pl.ds(i * 16, 16)] = zeros16

    pltpu.sync_copy(dst3.at[wid], idxbuf)

    @pl.loop(0, CPT)
    def _(j):
        for k in range(CHUNK // 16):
            idx = idxbuf[j, pl.ds(k * 16, 16)]
            plsc.addupdate_scatter(deg_local, [idx], ones16)

    pltpu.sync_copy(deg_local, degp.at[wid])


def _agg_body(xs_hbm, src3, dst3, outp, sbuf, dbuf, rows0, rows1, acc,
              gsem0, gsem1):
    cid, sid, wid = _wid()
    zeros16 = jnp.zeros((16,), jnp.float32)
    rows = (rows0, rows1)
    gsem = (gsem0, gsem1)

    @pl.loop(0, CHUNK)
    def _(i):
        for k in range(D // 16):
            rows0[i, pl.ds(k * 16, 16)] = zeros16

    for b in range(RPT // CHUNK):
        pltpu.sync_copy(rows0, acc.at[pl.ds(sid * RPT + b * CHUNK, CHUNK)])

    plsc.subcore_barrier()

    CH = CPT // 2
    for h in range(2):
        pltpu.sync_copy(src3.at[wid, pl.ds(h * CH, CH)], sbuf)
        pltpu.sync_copy(dst3.at[wid, pl.ds(h * CH, CH)], dbuf)

        @pl.loop(0, CH)
        def _(j):
            pltpu.async_copy(xs_hbm.at[sbuf.at[j]], rows0, gsem0).wait()
            pltpu.sync_copy(rows0, acc.at[dbuf.at[j]], add=True)

    plsc.subcore_barrier()
    pltpu.sync_copy(acc.at[pl.ds(sid * RPT, RPT)],
                    outp.at[cid].at[pl.ds(sid * RPT, RPT)])


def _make_sc_deg():
    return pl.kernel(
        _deg_body,
        out_type=jax.ShapeDtypeStruct((32, NP), jnp.float32),
        mesh=plsc.VectorSubcoreMesh(core_axis_name="c", subcore_axis_name="s"),
        compiler_params=pltpu.CompilerParams(needs_layout_passes=False),
        scratch_types=[
            pltpu.VMEM((CPT, CHUNK), jnp.int32),
            pltpu.VMEM((NP,), jnp.float32),
        ],
    )


def _make_sc_agg():
    return pl.kernel(
        _agg_body,
        out_type=jax.ShapeDtypeStruct((2, NP, D), jnp.float32),
        mesh=plsc.VectorSubcoreMesh(core_axis_name="c", subcore_axis_name="s"),
        compiler_params=pltpu.CompilerParams(needs_layout_passes=False),
        scratch_types=[
            pltpu.VMEM((CPT // 2, CHUNK), jnp.int32),
            pltpu.VMEM((CPT // 2, CHUNK), jnp.int32),
            pltpu.VMEM((CHUNK, D), jnp.float32),
            pltpu.VMEM((CHUNK, D), jnp.float32),
            pltpu.VMEM_SHARED((NP, D), jnp.float32),
            pltpu.SemaphoreType.DMA,
            pltpu.SemaphoreType.DMA,
        ],
    )


def _inv_col(degc):
    deg = jnp.sum(degc, axis=1, keepdims=True)
    return lax.rsqrt(jnp.maximum(deg, 1.0))


def _prescale_body(x_ref, degc_ref, xs_ref):
    xs_ref[...] = x_ref[...] * _inv_col(degc_ref[...])


def _mid_body(sp_ref, w_ref, degc_ref, hs_ref):
    s = sp_ref[0] + sp_ref[1]
    t = jnp.maximum(jnp.dot(s, w_ref[...], preferred_element_type=jnp.float32), 0.0)
    nrm = jnp.sqrt(jnp.sum(t * t, axis=1, keepdims=True))
    h = t / jnp.maximum(nrm, 1e-12)
    hs_ref[...] = h * _inv_col(degc_ref[...])


def _out_body(sp_ref, w_ref, degc_ref, o_ref):
    s = (sp_ref[0] + sp_ref[1]) * _inv_col(degc_ref[...])
    o_ref[...] = jnp.dot(s, w_ref[...], preferred_element_type=jnp.float32)


def kernel(x, edge_index, W1, W2):
    src = edge_index[0].astype(jnp.int32)
    dst = edge_index[1].astype(jnp.int32)
    pad = jnp.full((EP - E,), N, jnp.int32)
    srcp = jnp.concatenate([src, pad]).reshape(32, CPT, CHUNK)
    dstp = jnp.concatenate([dst, pad]).reshape(32, CPT, CHUNK)
    x_pad = jnp.pad(x, ((0, NP - N), (0, 0)))

    degp = _make_sc_deg()(dstp)
    degc = degp.T

    xs = pl.pallas_call(
        _prescale_body,
        out_shape=jax.ShapeDtypeStruct((NP, D), jnp.float32),
    )(x_pad, degc)

    s1 = _make_sc_agg()(xs, srcp, dstp)

    hs = pl.pallas_call(
        _mid_body,
        out_shape=jax.ShapeDtypeStruct((NP, D), jnp.float32),
    )(s1, W1, degc)

    s2 = _make_sc_agg()(hs, srcp, dstp)

    outp = pl.pallas_call(
        _out_body,
        out_shape=jax.ShapeDtypeStruct((NP, D), jnp.float32),
    )(s2, W2, degc)

    return outp[:N]


# speedup vs baseline: 1.2086x; 1.2086x over previous
import functools

import jax
import jax.numpy as jnp
from jax import lax
from jax.experimental import pallas as pl
from jax.experimental.pallas import tpu as pltpu
from jax.experimental.pallas import tpu_sc as plsc

N = 10000
D = 128
E = 320000
NP = 10240
CHUNK = 128
CPT = 80
EPT = CPT * CHUNK
EP = 32 * EPT
RPT = NP // 16


def _wid():
    cid = lax.axis_index("c")
    sid = lax.axis_index("s")
    return cid, sid, sid * 2 + cid


def _deg_body(dst3, degp, idxbuf, deg_local):
    cid, sid, wid = _wid()
    zeros16 = jnp.zeros((16,), jnp.float32)
    ones16 = jnp.ones((16,), jnp.float32)

    @pl.loop(0, NP // 16)
    def _(i):
        deg_local[pl.ds(i * 16, 16)] = zeros16

    pltpu.sync_copy(dst3.at[wid], idxbuf)

    @pl.loop(0, CPT)
    def _(j):
        for k in range(CHUNK // 16):
            idx = idxbuf[j, pl.ds(k * 16, 16)]
            plsc.addupdate_scatter(deg_local, [idx], ones16)

    pltpu.sync_copy(deg_local, degp.at[wid])


def _agg_body(xs_hbm, src1, dst1, outp,
              sidx0, sidx1, didx0, didx1, rows0, rows1, acc,
              gs0, gs1, is0, is1, id0, id1):
    cid, sid, wid = _wid()
    zeros16 = jnp.zeros((16,), jnp.float32)
    sidx = (sidx0, sidx1)
    didx = (didx0, didx1)
    rows = (rows0, rows1)
    gsem = (gs0, gs1)
    isem = (is0, is1)
    idsem = (id0, id1)
    base = wid * EPT

    @pl.loop(0, CHUNK)
    def _(i):
        for k in range(D // 16):
            rows0[i, pl.ds(k * 16, 16)] = zeros16

    for b in range(RPT // CHUNK):
        pltpu.sync_copy(rows0, acc.at[pl.ds(sid * RPT + b * CHUNK, CHUNK)])
    plsc.subcore_barrier()

    def sslice(arr, c):
        return arr.at[pl.ds(base + c * CHUNK, CHUNK)]

    pltpu.sync_copy(sslice(src1, 0), sidx0)
    pltpu.sync_copy(sslice(dst1, 0), didx0)
    pltpu.async_copy(xs_hbm.at[sidx0], rows0, gs0)
    pltpu.async_copy(sslice(src1, 1), sidx1, is1)
    pltpu.async_copy(sslice(dst1, 1), didx1, id1)

    @pl.loop(0, CPT, step=2)
    def _(i):
        for b in range(2):
            j = i + b
            nb = 1 - b
            pltpu.make_async_copy(sslice(src1, j + 1), sidx[nb],
                                  isem[nb]).wait()
            pltpu.make_async_copy(sslice(dst1, j + 1), didx[nb],
                                  idsem[nb]).wait()
            pltpu.async_copy(xs_hbm.at[sidx[nb]], rows[nb], gsem[nb])
            pltpu.make_async_copy(xs_hbm.at[sidx[b]], rows[b], gsem[b]).wait()
            pltpu.sync_copy(rows[b], acc.at[didx[b]], add=True)
            pltpu.async_copy(sslice(src1, j + 2), sidx[b], isem[b])
            pltpu.async_copy(sslice(dst1, j + 2), didx[b], idsem[b])

    pltpu.make_async_copy(sslice(src1, CPT + 1), sidx1, is1).wait()
    pltpu.make_async_copy(sslice(dst1, CPT + 1), didx1, id1).wait()
    pltpu.make_async_copy(xs_hbm.at[sidx0], rows0, gs0).wait()

    plsc.subcore_barrier()
    pltpu.sync_copy(acc.at[pl.ds(sid * RPT, RPT)],
                    outp.at[cid].at[pl.ds(sid * RPT, RPT)])


def _make_sc_deg():
    return pl.kernel(
        _deg_body,
        out_type=jax.ShapeDtypeStruct((32, NP), jnp.float32),
        mesh=plsc.VectorSubcoreMesh(core_axis_name="c", subcore_axis_name="s"),
        compiler_params=pltpu.CompilerParams(needs_layout_passes=False),
        scratch_types=[
            pltpu.VMEM((CPT, CHUNK), jnp.int32),
            pltpu.VMEM((NP,), jnp.float32),
        ],
    )


def _make_sc_agg():
    return pl.kernel(
        _agg_body,
        out_type=jax.ShapeDtypeStruct((2, NP, D), jnp.float32),
        mesh=plsc.VectorSubcoreMesh(core_axis_name="c", subcore_axis_name="s"),
        compiler_params=pltpu.CompilerParams(needs_layout_passes=False),
        scratch_types=[
            pltpu.VMEM((CHUNK,), jnp.int32),
            pltpu.VMEM((CHUNK,), jnp.int32),
            pltpu.VMEM((CHUNK,), jnp.int32),
            pltpu.VMEM((CHUNK,), jnp.int32),
            pltpu.VMEM((CHUNK, D), jnp.float32),
            pltpu.VMEM((CHUNK, D), jnp.float32),
            pltpu.VMEM_SHARED((NP, D), jnp.float32),
        ] + [pltpu.SemaphoreType.DMA] * 6,
    )


def _inv_col(degc):
    deg = jnp.sum(degc, axis=1, keepdims=True)
    return lax.rsqrt(jnp.maximum(deg, 1.0))


def _prescale_body(x_ref, degc_ref, xs_ref):
    xs_ref[...] = x_ref[...] * _inv_col(degc_ref[...])


def _mid_body(sp_ref, w_ref, degc_ref, hs_ref):
    s = sp_ref[0] + sp_ref[1]
    t = jnp.maximum(jnp.dot(s, w_ref[...], preferred_element_type=jnp.float32), 0.0)
    nrm = jnp.sqrt(jnp.sum(t * t, axis=1, keepdims=True))
    h = t / jnp.maximum(nrm, 1e-12)
    hs_ref[...] = h * _inv_col(degc_ref[...])


def _out_body(sp_ref, w_ref, degc_ref, o_ref):
    s = (sp_ref[0] + sp_ref[1]) * _inv_col(degc_ref[...])
    o_ref[...] = jnp.dot(s, w_ref[...], preferred_element_type=jnp.float32)


def kernel(x, edge_index, W1, W2):
    src = edge_index[0].astype(jnp.int32)
    dst = edge_index[1].astype(jnp.int32)
    pad = jnp.full((EP + 2 * CHUNK - E,), N, jnp.int32)
    srcp = jnp.concatenate([src, pad])
    dstp = jnp.concatenate([dst, pad])
    x_pad = jnp.pad(x, ((0, NP - N), (0, 0)))

    degp = _make_sc_deg()(dstp[:EP].reshape(32, CPT, CHUNK))
    degc = degp.T

    xs = pl.pallas_call(
        _prescale_body,
        out_shape=jax.ShapeDtypeStruct((NP, D), jnp.float32),
    )(x_pad, degc)

    s1 = _make_sc_agg()(xs, srcp, dstp)

    hs = pl.pallas_call(
        _mid_body,
        out_shape=jax.ShapeDtypeStruct((NP, D), jnp.float32),
    )(s1, W1, degc)

    s2 = _make_sc_agg()(hs, srcp, dstp)

    outp = pl.pallas_call(
        _out_body,
        out_shape=jax.ShapeDtypeStruct((NP, D), jnp.float32),
    )(s2, W2, degc)

    return outp[:N]
